# baseline (speedup 1.0000x reference)
import functools

import jax
import jax.numpy as jnp
from jax import lax
from jax.experimental import pallas as pl
from jax.experimental.pallas import tpu as pltpu
from jax.experimental.pallas import tpu_sc as plsc

B, S, D, F, E = 2, 2048, 768, 3072, 8
CAP = 400
N = B * S
EROWS = E * CAP
RBLK = 400
RB = -(-N // RBLK)
TOT = EROWS + RB * RBLK
BLK = 512
NB = N // BLK
FBLK = 768
FB = F // FBLK
NW = 32
CHUNK = N // NW


def _route_body(x_ref, wrt_ref, br_ref, slot_ref, ovcnt_ref, base_ref, ovbase_ref):
    i = pl.program_id(0)

    @pl.when(i == 0)
    def _init():
        base_ref[...] = jnp.zeros_like(base_ref)
        ovbase_ref[...] = jnp.zeros_like(ovbase_ref)

    xb = x_ref[...]
    logits = jnp.dot(xb, wrt_ref[...], preferred_element_type=jnp.float32)
    logits = logits + br_ref[...]

    maxv = jnp.max(logits, axis=1, keepdims=True)
    cols = lax.broadcasted_iota(jnp.int32, (BLK, E), 1).astype(jnp.float32)
    eidf = jnp.min(jnp.where(logits >= maxv, cols, float(E)), axis=1,
                   keepdims=True)
    onehot = (cols == eidf).astype(jnp.float32)

    rws = lax.broadcasted_iota(jnp.int32, (BLK, BLK), 0)
    cls = lax.broadcasted_iota(jnp.int32, (BLK, BLK), 1)
    ltri = (rws >= cls).astype(jnp.float32)
    csum = jnp.dot(ltri, onehot, preferred_element_type=jnp.float32)

    rank = jnp.sum(onehot * (base_ref[...] + csum), axis=1,
                   keepdims=True) - 1.0
    keep = rank < float(CAP)
    ov = jnp.where(keep, 0.0, 1.0)
    ovcsum = jnp.dot(ltri, ov, preferred_element_type=jnp.float32)
    ovrank = ovbase_ref[...] + ovcsum - 1.0

    slotf = jnp.where(keep, eidf * float(CAP) + rank,
                      float(EROWS) + ovrank)
    slot_ref[...] = slotf.astype(jnp.int32)

    base_ref[...] = base_ref[...] + jnp.sum(onehot, axis=0, keepdims=True)
    ovbase_ref[...] = ovbase_ref[...] + jnp.sum(ov, axis=0, keepdims=True)

    @pl.when(i == NB - 1)
    def _fin():
        ovcnt_ref[...] = ovbase_ref[...].astype(jnp.int32)


def _route_call(xf, wrt, br2, interpret=False):
    return pl.pallas_call(
        _route_body,
        grid=(NB,),
        in_specs=[
            pl.BlockSpec((BLK, D), lambda i: (i, 0)),
            pl.BlockSpec((D, E), lambda i: (0, 0)),
            pl.BlockSpec((1, E), lambda i: (0, 0)),
        ],
        out_specs=[
            pl.BlockSpec((BLK, 1), lambda i: (i, 0)),
            pl.BlockSpec((1, 1), lambda i: (0, 0)),
        ],
        out_shape=[
            jax.ShapeDtypeStruct((N, 1), jnp.int32),
            jax.ShapeDtypeStruct((1, 1), jnp.int32),
        ],
        scratch_shapes=[
            pltpu.VMEM((1, E), jnp.float32),
            pltpu.VMEM((1, 1), jnp.float32),
        ],
        interpret=interpret,
    )(xf, wrt, br2)


def _dispatch_body(x_hbm, slot_hbm, out_hbm, idx_v, rows_v, sem):
    wid = lax.axis_index("s") * 2 + lax.axis_index("c")
    base = wid * CHUNK
    pltpu.sync_copy(slot_hbm.at[pl.ds(base, CHUNK)], idx_v)
    pltpu.sync_copy(x_hbm.at[pl.ds(base, CHUNK)], rows_v)
    pltpu.async_copy(rows_v, out_hbm.at[idx_v], sem).wait()


def _dispatch_call(xf, slot):
    mesh = plsc.VectorSubcoreMesh(core_axis_name="c", subcore_axis_name="s",
                                  num_cores=2, num_subcores=16)
    return pl.kernel(
        _dispatch_body,
        out_type=jax.ShapeDtypeStruct((TOT, D), jnp.float32),
        mesh=mesh,
        scratch_types=[
            pltpu.VMEM((CHUNK,), jnp.int32),
            pltpu.VMEM((CHUNK, D), jnp.float32),
            pltpu.SemaphoreType.DMA,
        ],
    )(xf, slot)


G = E + RB


def _na_of(cnt):
    return (cnt[0] + RBLK - 1) // RBLK


def _row_idx(g, fb, cnt):
    return (jnp.minimum(g, E - 1 + _na_of(cnt)), 0)


def _fb_eff(g, fb, cnt):
    na = _na_of(cnt)
    active = jnp.logical_and(g >= E, g < E + na)
    return jnp.where(g < E, 0,
                     jnp.where(active, fb, jnp.where(na > 0, FB - 1, 0)))


def _moe_body(cnt_ref, x_ref, w1_ref, b1_ref, w2_ref, b2_ref,
              wo1_ref, bo1_ref, wo2_ref, bo2_ref, out_ref, acc_ref):
    g = pl.program_id(0)
    fb = pl.program_id(1)
    na = (cnt_ref[0] + RBLK - 1) // RBLK

    def ffn_step(w1, b1, w2, b2):
        h = jnp.dot(x_ref[...], w1, preferred_element_type=jnp.float32)
        h = jnp.maximum(h + b1, 0.0)
        part = jnp.dot(h, w2, preferred_element_type=jnp.float32)

        @pl.when(fb == 0)
        def _first():
            acc_ref[...] = part + b2

        @pl.when(fb > 0)
        def _rest():
            acc_ref[...] = acc_ref[...] + part

        @pl.when(fb == FB - 1)
        def _last():
            out_ref[...] = acc_ref[...]

    @pl.when(g < E)
    def _expert():
        ffn_step(w1_ref[0], b1_ref[0], w2_ref[0], b2_ref[0])

    @pl.when(jnp.logical_and(g >= E, g - E < na))
    def _ovf():
        ffn_step(wo1_ref[...], bo1_ref[...], wo2_ref[...], bo2_ref[...])


def _moe_ffn_call(ovcnt1, disp, W1, b1, W2, b2, Wo1, bo1, Wo2, bo2,
                  interpret=False):
    grid_spec = pltpu.PrefetchScalarGridSpec(
        num_scalar_prefetch=1,
        grid=(G, FB),
        in_specs=[
            pl.BlockSpec((RBLK, D), _row_idx),
            pl.BlockSpec((1, D, FBLK),
                         lambda g, fb, c: (jnp.minimum(g, E - 1), 0,
                                           jnp.where(g < E, fb, FB - 1))),
            pl.BlockSpec((1, 1, FBLK),
                         lambda g, fb, c: (jnp.minimum(g, E - 1), 0,
                                           jnp.where(g < E, fb, FB - 1))),
            pl.BlockSpec((1, FBLK, D),
                         lambda g, fb, c: (jnp.minimum(g, E - 1),
                                           jnp.where(g < E, fb, FB - 1), 0)),
            pl.BlockSpec((1, 1, D),
                         lambda g, fb, c: (jnp.minimum(g, E - 1), 0, 0)),
            pl.BlockSpec((D, FBLK), lambda g, fb, c: (0, _fb_eff(g, fb, c))),
            pl.BlockSpec((1, FBLK), lambda g, fb, c: (0, _fb_eff(g, fb, c))),
            pl.BlockSpec((FBLK, D), lambda g, fb, c: (_fb_eff(g, fb, c), 0)),
            pl.BlockSpec((1, D), lambda g, fb, c: (0, 0)),
        ],
        out_specs=pl.BlockSpec((RBLK, D), _row_idx),
        scratch_shapes=[pltpu.VMEM((RBLK, D), jnp.float32)],
    )
    return pl.pallas_call(
        _moe_body,
        grid_spec=grid_spec,
        out_shape=jax.ShapeDtypeStruct((TOT, D), jnp.float32),
        interpret=interpret,
    )(ovcnt1, disp, W1, b1.reshape(E, 1, F), W2, b2.reshape(E, 1, D),
      Wo1, bo1.reshape(1, F), Wo2, bo2.reshape(1, D))


def _combine_body(ffn_hbm, slot_hbm, out_hbm, idx_v, rows_v, sem):
    wid = lax.axis_index("s") * 2 + lax.axis_index("c")
    base = wid * CHUNK
    pltpu.sync_copy(slot_hbm.at[pl.ds(base, CHUNK)], idx_v)
    pltpu.async_copy(ffn_hbm.at[idx_v], rows_v, sem).wait()
    pltpu.sync_copy(rows_v, out_hbm.at[pl.ds(base, CHUNK)])


def _combine_call(ffn, slot):
    mesh = plsc.VectorSubcoreMesh(core_axis_name="c", subcore_axis_name="s",
                                  num_cores=2, num_subcores=16)
    return pl.kernel(
        _combine_body,
        out_type=jax.ShapeDtypeStruct((N, D), jnp.float32),
        mesh=mesh,
        scratch_types=[
            pltpu.VMEM((CHUNK,), jnp.int32),
            pltpu.VMEM((CHUNK, D), jnp.float32),
            pltpu.SemaphoreType.DMA,
        ],
    )(ffn, slot)


def kernel(x, Wr, br, W1, b1, W2, b2, Wo1, bo1, Wo2, bo2):
    xf = x.reshape(N, D)
    slot2, ovcnt = _route_call(xf, Wr.T, br.reshape(1, E))
    slot = slot2.reshape(N)
    disp = _dispatch_call(xf, slot)
    ffn = _moe_ffn_call(ovcnt.reshape(1), disp, W1, b1, W2, b2,
                        Wo1, bo1, Wo2, bo2)
    out = _combine_call(ffn, slot)
    return out.reshape(B, S, D)


# speedup vs baseline: 1.1179x; 1.1179x over previous
import functools

import jax
import jax.numpy as jnp
from jax import lax
from jax.experimental import pallas as pl
from jax.experimental.pallas import tpu as pltpu
from jax.experimental.pallas import tpu_sc as plsc

B, S, D, F, E = 2, 2048, 768, 3072, 8
CAP = 400
N = B * S
EROWS = E * CAP
RBLK = 400
RB = -(-N // RBLK)
TOT = EROWS + RB * RBLK
BLK = 512
NB = N // BLK
FBLK = 1536
FB = F // FBLK
NW = 32
CHUNK = N // NW


def _route_body(x_ref, wrt_ref, br_ref, slot_ref, ovcnt_ref, base_ref, ovbase_ref):
    i = pl.program_id(0)

    @pl.when(i == 0)
    def _init():
        base_ref[...] = jnp.zeros_like(base_ref)
        ovbase_ref[...] = jnp.zeros_like(ovbase_ref)

    xb = x_ref[...]
    logits = jnp.dot(xb, wrt_ref[...], preferred_element_type=jnp.float32)
    logits = logits + br_ref[...]

    maxv = jnp.max(logits, axis=1, keepdims=True)
    cols = lax.broadcasted_iota(jnp.int32, (BLK, E), 1).astype(jnp.float32)
    eidf = jnp.min(jnp.where(logits >= maxv, cols, float(E)), axis=1,
                   keepdims=True)
    onehot = (cols == eidf).astype(jnp.float32)

    rws = lax.broadcasted_iota(jnp.int32, (BLK, BLK), 0)
    cls = lax.broadcasted_iota(jnp.int32, (BLK, BLK), 1)
    ltri = (rws >= cls).astype(jnp.float32)
    csum = jnp.dot(ltri, onehot, preferred_element_type=jnp.float32)

    rank = jnp.sum(onehot * (base_ref[...] + csum), axis=1,
                   keepdims=True) - 1.0
    keep = rank < float(CAP)
    ov = jnp.where(keep, 0.0, 1.0)
    ovcsum = jnp.dot(ltri, ov, preferred_element_type=jnp.float32)
    ovrank = ovbase_ref[...] + ovcsum - 1.0

    slotf = jnp.where(keep, eidf * float(CAP) + rank,
                      float(EROWS) + ovrank)
    slot_ref[...] = slotf.astype(jnp.int32)

    base_ref[...] = base_ref[...] + jnp.sum(onehot, axis=0, keepdims=True)
    ovbase_ref[...] = ovbase_ref[...] + jnp.sum(ov, axis=0, keepdims=True)

    @pl.when(i == NB - 1)
    def _fin():
        ovcnt_ref[...] = ovbase_ref[...].astype(jnp.int32)


def _route_call(xf, wrt, br2, interpret=False):
    return pl.pallas_call(
        _route_body,
        grid=(NB,),
        in_specs=[
            pl.BlockSpec((BLK, D), lambda i: (i, 0)),
            pl.BlockSpec((D, E), lambda i: (0, 0)),
            pl.BlockSpec((1, E), lambda i: (0, 0)),
        ],
        out_specs=[
            pl.BlockSpec((BLK, 1), lambda i: (i, 0)),
            pl.BlockSpec((1, 1), lambda i: (0, 0)),
        ],
        out_shape=[
            jax.ShapeDtypeStruct((N, 1), jnp.int32),
            jax.ShapeDtypeStruct((1, 1), jnp.int32),
        ],
        scratch_shapes=[
            pltpu.VMEM((1, E), jnp.float32),
            pltpu.VMEM((1, 1), jnp.float32),
        ],
        interpret=interpret,
    )(xf, wrt, br2)


HALF = CHUNK // 2


def _dispatch_body(x_hbm, slot_hbm, out_hbm, idx0_v, idx1_v, rows0_v, rows1_v,
                   sem_r, sem_w):
    wid = lax.axis_index("s") * 2 + lax.axis_index("c")
    base = wid * CHUNK
    pltpu.sync_copy(slot_hbm.at[pl.ds(base, HALF)], idx0_v)
    pltpu.sync_copy(slot_hbm.at[pl.ds(base + HALF, HALF)], idx1_v)
    r0 = pltpu.async_copy(x_hbm.at[pl.ds(base, HALF)], rows0_v, sem_r)
    r1 = pltpu.async_copy(x_hbm.at[pl.ds(base + HALF, HALF)], rows1_v, sem_r)
    r0.wait()
    w0 = pltpu.async_copy(rows0_v, out_hbm.at[idx0_v], sem_w)
    r1.wait()
    w1 = pltpu.async_copy(rows1_v, out_hbm.at[idx1_v], sem_w)
    w0.wait()
    w1.wait()


def _dispatch_call(xf, slot):
    mesh = plsc.VectorSubcoreMesh(core_axis_name="c", subcore_axis_name="s",
                                  num_cores=2, num_subcores=16)
    return pl.kernel(
        _dispatch_body,
        out_type=jax.ShapeDtypeStruct((TOT, D), jnp.float32),
        mesh=mesh,
        scratch_types=[
            pltpu.VMEM((HALF,), jnp.int32),
            pltpu.VMEM((HALF,), jnp.int32),
            pltpu.VMEM((HALF, D), jnp.float32),
            pltpu.VMEM((HALF, D), jnp.float32),
            pltpu.SemaphoreType.DMA,
            pltpu.SemaphoreType.DMA,
        ],
    )(xf, slot)


G = E + RB


def _na_of(cnt):
    return (cnt[0] + RBLK - 1) // RBLK


def _row_idx(g, fb, cnt):
    return (jnp.minimum(g, E - 1 + _na_of(cnt)), 0)


def _fb_eff(g, fb, cnt):
    na = _na_of(cnt)
    active = jnp.logical_and(g >= E, g < E + na)
    return jnp.where(g < E, 0,
                     jnp.where(active, fb, jnp.where(na > 0, FB - 1, 0)))


def _moe_body(cnt_ref, x_ref, w1_ref, b1_ref, w2_ref, b2_ref,
              wo1_ref, bo1_ref, wo2_ref, bo2_ref, out_ref, acc_ref):
    g = pl.program_id(0)
    fb = pl.program_id(1)
    na = (cnt_ref[0] + RBLK - 1) // RBLK

    def ffn_step(w1, b1, w2, b2):
        h = jnp.dot(x_ref[...], w1, preferred_element_type=jnp.float32)
        h = jnp.maximum(h + b1, 0.0)
        part = jnp.dot(h, w2, preferred_element_type=jnp.float32)

        @pl.when(fb == 0)
        def _first():
            acc_ref[...] = part + b2

        @pl.when(fb > 0)
        def _rest():
            acc_ref[...] = acc_ref[...] + part

        @pl.when(fb == FB - 1)
        def _last():
            out_ref[...] = acc_ref[...]

    @pl.when(g < E)
    def _expert():
        ffn_step(w1_ref[0], b1_ref[0], w2_ref[0], b2_ref[0])

    @pl.when(jnp.logical_and(g >= E, g - E < na))
    def _ovf():
        ffn_step(wo1_ref[...], bo1_ref[...], wo2_ref[...], bo2_ref[...])


def _moe_ffn_call(ovcnt1, disp, W1, b1, W2, b2, Wo1, bo1, Wo2, bo2,
                  interpret=False):
    grid_spec = pltpu.PrefetchScalarGridSpec(
        num_scalar_prefetch=1,
        grid=(G, FB),
        in_specs=[
            pl.BlockSpec((RBLK, D), _row_idx),
            pl.BlockSpec((1, D, FBLK),
                         lambda g, fb, c: (jnp.minimum(g, E - 1), 0,
                                           jnp.where(g < E, fb, FB - 1))),
            pl.BlockSpec((1, 1, FBLK),
                         lambda g, fb, c: (jnp.minimum(g, E - 1), 0,
                                           jnp.where(g < E, fb, FB - 1))),
            pl.BlockSpec((1, FBLK, D),
                         lambda g, fb, c: (jnp.minimum(g, E - 1),
                                           jnp.where(g < E, fb, FB - 1), 0)),
            pl.BlockSpec((1, 1, D),
                         lambda g, fb, c: (jnp.minimum(g, E - 1), 0, 0)),
            pl.BlockSpec((D, FBLK), lambda g, fb, c: (0, _fb_eff(g, fb, c))),
            pl.BlockSpec((1, FBLK), lambda g, fb, c: (0, _fb_eff(g, fb, c))),
            pl.BlockSpec((FBLK, D), lambda g, fb, c: (_fb_eff(g, fb, c), 0)),
            pl.BlockSpec((1, D), lambda g, fb, c: (0, 0)),
        ],
        out_specs=pl.BlockSpec((RBLK, D), _row_idx),
        scratch_shapes=[pltpu.VMEM((RBLK, D), jnp.float32)],
    )
    return pl.pallas_call(
        _moe_body,
        grid_spec=grid_spec,
        out_shape=jax.ShapeDtypeStruct((TOT, D), jnp.float32),
        interpret=interpret,
    )(ovcnt1, disp, W1, b1.reshape(E, 1, F), W2, b2.reshape(E, 1, D),
      Wo1, bo1.reshape(1, F), Wo2, bo2.reshape(1, D))


def _combine_body(ffn_hbm, slot_hbm, out_hbm, idx0_v, idx1_v, rows0_v, rows1_v,
                  sem_r, sem_w):
    wid = lax.axis_index("s") * 2 + lax.axis_index("c")
    base = wid * CHUNK
    pltpu.sync_copy(slot_hbm.at[pl.ds(base, HALF)], idx0_v)
    pltpu.sync_copy(slot_hbm.at[pl.ds(base + HALF, HALF)], idx1_v)
    g0 = pltpu.async_copy(ffn_hbm.at[idx0_v], rows0_v, sem_r)
    g1 = pltpu.async_copy(ffn_hbm.at[idx1_v], rows1_v, sem_r)
    g0.wait()
    w0 = pltpu.async_copy(rows0_v, out_hbm.at[pl.ds(base, HALF)], sem_w)
    g1.wait()
    w1 = pltpu.async_copy(rows1_v, out_hbm.at[pl.ds(base + HALF, HALF)], sem_w)
    w0.wait()
    w1.wait()


def _combine_call(ffn, slot):
    mesh = plsc.VectorSubcoreMesh(core_axis_name="c", subcore_axis_name="s",
                                  num_cores=2, num_subcores=16)
    return pl.kernel(
        _combine_body,
        out_type=jax.ShapeDtypeStruct((N, D), jnp.float32),
        mesh=mesh,
        scratch_types=[
            pltpu.VMEM((HALF,), jnp.int32),
            pltpu.VMEM((HALF,), jnp.int32),
            pltpu.VMEM((HALF, D), jnp.float32),
            pltpu.VMEM((HALF, D), jnp.float32),
            pltpu.SemaphoreType.DMA,
            pltpu.SemaphoreType.DMA,
        ],
    )(ffn, slot)


def kernel(x, Wr, br, W1, b1, W2, b2, Wo1, bo1, Wo2, bo2):
    xf = x.reshape(N, D)
    slot2, ovcnt = _route_call(xf, Wr.T, br.reshape(1, E))
    slot = slot2.reshape(N)
    disp = _dispatch_call(xf, slot)
    ffn = _moe_ffn_call(ovcnt.reshape(1), disp, W1, b1, W2, b2,
                        Wo1, bo1, Wo2, bo2)
    out = _combine_call(ffn, slot)
    return out.reshape(B, S, D)


# speedup vs baseline: 1.1878x; 1.0625x over previous
import functools

import jax
import jax.numpy as jnp
from jax import lax
from jax.experimental import pallas as pl
from jax.experimental.pallas import tpu as pltpu
from jax.experimental.pallas import tpu_sc as plsc

B, S, D, F, E = 2, 2048, 768, 3072, 8
CAP = 400
N = B * S
EROWS = E * CAP
RBLK = 400
RB = -(-N // RBLK)
TOT = EROWS + RB * RBLK
BLK = 512
NB = N // BLK
FBLK = 1536
FB = F // FBLK
NW = 32
CHUNK = N // NW


def _route_body(x_ref, wrt_ref, br_ref, slot_ref, ovcnt_ref, base_ref, ovbase_ref):
    i = pl.program_id(0)

    @pl.when(i == 0)
    def _init():
        base_ref[...] = jnp.zeros_like(base_ref)
        ovbase_ref[...] = jnp.zeros_like(ovbase_ref)

    xb = x_ref[...]
    logits = jnp.dot(xb, wrt_ref[...], preferred_element_type=jnp.float32)
    logits = logits + br_ref[...]

    maxv = jnp.max(logits, axis=1, keepdims=True)
    cols = lax.broadcasted_iota(jnp.int32, (BLK, E), 1).astype(jnp.float32)
    eidf = jnp.min(jnp.where(logits >= maxv, cols, float(E)), axis=1,
                   keepdims=True)
    onehot = (cols == eidf).astype(jnp.float32)

    rws = lax.broadcasted_iota(jnp.int32, (BLK, BLK), 0)
    cls = lax.broadcasted_iota(jnp.int32, (BLK, BLK), 1)
    ltri = (rws >= cls).astype(jnp.float32)
    csum = jnp.dot(ltri, onehot, preferred_element_type=jnp.float32)

    rank = jnp.sum(onehot * (base_ref[...] + csum), axis=1,
                   keepdims=True) - 1.0
    keep = rank < float(CAP)
    ov = jnp.where(keep, 0.0, 1.0)
    ovcsum = jnp.dot(ltri, ov, preferred_element_type=jnp.float32)
    ovrank = ovbase_ref[...] + ovcsum - 1.0

    slotf = jnp.where(keep, eidf * float(CAP) + rank,
                      float(EROWS) + ovrank)
    slot_ref[...] = slotf.astype(jnp.int32)

    base_ref[...] = base_ref[...] + jnp.sum(onehot, axis=0, keepdims=True)
    ovbase_ref[...] = ovbase_ref[...] + jnp.sum(ov, axis=0, keepdims=True)

    @pl.when(i == NB - 1)
    def _fin():
        ovcnt_ref[...] = ovbase_ref[...].astype(jnp.int32)


def _route_call(xf, wrt, br2, interpret=False):
    return pl.pallas_call(
        _route_body,
        grid=(NB,),
        in_specs=[
            pl.BlockSpec((BLK, D), lambda i: (i, 0)),
            pl.BlockSpec((D, E), lambda i: (0, 0)),
            pl.BlockSpec((1, E), lambda i: (0, 0)),
        ],
        out_specs=[
            pl.BlockSpec((BLK, 1), lambda i: (i, 0)),
            pl.BlockSpec((1, 1), lambda i: (0, 0)),
        ],
        out_shape=[
            jax.ShapeDtypeStruct((N, 1), jnp.int32),
            jax.ShapeDtypeStruct((1, 1), jnp.int32),
        ],
        scratch_shapes=[
            pltpu.VMEM((1, E), jnp.float32),
            pltpu.VMEM((1, 1), jnp.float32),
        ],
        interpret=interpret,
    )(xf, wrt, br2)


HALF = CHUNK // 2


def _dispatch_body(x_hbm, slot_hbm, out_hbm, idx0_v, idx1_v, rows0_v, rows1_v,
                   sem_r, sem_w):
    wid = lax.axis_index("s") * 2 + lax.axis_index("c")
    base = wid * CHUNK
    pltpu.sync_copy(slot_hbm.at[pl.ds(base, HALF)], idx0_v)
    pltpu.sync_copy(slot_hbm.at[pl.ds(base + HALF, HALF)], idx1_v)
    r0 = pltpu.async_copy(x_hbm.at[pl.ds(base, HALF)], rows0_v, sem_r)
    r1 = pltpu.async_copy(x_hbm.at[pl.ds(base + HALF, HALF)], rows1_v, sem_r)
    r0.wait()
    w0 = pltpu.async_copy(rows0_v, out_hbm.at[idx0_v], sem_w)
    r1.wait()
    w1 = pltpu.async_copy(rows1_v, out_hbm.at[idx1_v], sem_w)
    w0.wait()
    w1.wait()


def _dispatch_call(xf, slot):
    mesh = plsc.VectorSubcoreMesh(core_axis_name="c", subcore_axis_name="s",
                                  num_cores=2, num_subcores=16)
    return pl.kernel(
        _dispatch_body,
        out_type=jax.ShapeDtypeStruct((TOT, D), jnp.float32),
        mesh=mesh,
        scratch_types=[
            pltpu.VMEM((HALF,), jnp.int32),
            pltpu.VMEM((HALF,), jnp.int32),
            pltpu.VMEM((HALF, D), jnp.float32),
            pltpu.VMEM((HALF, D), jnp.float32),
            pltpu.SemaphoreType.DMA,
            pltpu.SemaphoreType.DMA,
        ],
    )(xf, slot)


def _ffn_body(x_ref, w1_ref, b1_ref, w2_ref, b2_ref, out_ref):
    h = jnp.dot(x_ref[...], w1_ref[0], preferred_element_type=jnp.float32)
    h = jnp.maximum(h + b1_ref[0], 0.0)
    out_ref[...] = jnp.dot(h, w2_ref[0],
                           preferred_element_type=jnp.float32) + b2_ref[0]


def _ffn_call(disp, W1, b1, W2, b2, interpret=False):
    return pl.pallas_call(
        _ffn_body,
        grid=(E,),
        in_specs=[
            pl.BlockSpec((CAP, D), lambda e: (e, 0)),
            pl.BlockSpec((1, D, F), lambda e: (e, 0, 0)),
            pl.BlockSpec((1, 1, F), lambda e: (e, 0, 0)),
            pl.BlockSpec((1, F, D), lambda e: (e, 0, 0)),
            pl.BlockSpec((1, 1, D), lambda e: (e, 0, 0)),
        ],
        out_specs=pl.BlockSpec((CAP, D), lambda e: (e, 0)),
        out_shape=jax.ShapeDtypeStruct((TOT, D), jnp.float32),
        interpret=interpret,
    )(disp, W1, b1.reshape(E, 1, F), W2, b2.reshape(E, 1, D))


def _ovf_row_idx(rb, cnt):
    na = (cnt[0] + RBLK - 1) // RBLK
    return (E + jnp.minimum(rb, jnp.maximum(na, 1) - 1), 0)


def _ovf_body(cnt_ref, x_ref, wo1_ref, bo1_ref, wo2_ref, bo2_ref, _, out_ref):
    rb = pl.program_id(0)

    @pl.when(rb * RBLK < cnt_ref[0])
    def _go():
        h = jnp.dot(x_ref[...], wo1_ref[...], preferred_element_type=jnp.float32)
        h = jnp.maximum(h + bo1_ref[...], 0.0)
        out_ref[...] = jnp.dot(h, wo2_ref[...],
                               preferred_element_type=jnp.float32) + bo2_ref[...]


def _ovf_call(ovcnt1, disp, ffn, Wo1, bo1, Wo2, bo2, interpret=False):
    grid_spec = pltpu.PrefetchScalarGridSpec(
        num_scalar_prefetch=1,
        grid=(RB,),
        in_specs=[
            pl.BlockSpec((RBLK, D), _ovf_row_idx),
            pl.BlockSpec((D, F), lambda rb, c: (0, 0)),
            pl.BlockSpec((1, F), lambda rb, c: (0, 0)),
            pl.BlockSpec((F, D), lambda rb, c: (0, 0)),
            pl.BlockSpec((1, D), lambda rb, c: (0, 0)),
            pl.BlockSpec(memory_space=pl.ANY),
        ],
        out_specs=pl.BlockSpec((RBLK, D), _ovf_row_idx),
    )
    return pl.pallas_call(
        _ovf_body,
        grid_spec=grid_spec,
        out_shape=jax.ShapeDtypeStruct((TOT, D), jnp.float32),
        input_output_aliases={6: 0},
        interpret=interpret,
    )(ovcnt1, disp, Wo1, bo1.reshape(1, F), Wo2, bo2.reshape(1, D), ffn)


def _combine_body(ffn_hbm, slot_hbm, out_hbm, idx0_v, idx1_v, rows0_v, rows1_v,
                  sem_r, sem_w):
    wid = lax.axis_index("s") * 2 + lax.axis_index("c")
    base = wid * CHUNK
    pltpu.sync_copy(slot_hbm.at[pl.ds(base, HALF)], idx0_v)
    pltpu.sync_copy(slot_hbm.at[pl.ds(base + HALF, HALF)], idx1_v)
    g0 = pltpu.async_copy(ffn_hbm.at[idx0_v], rows0_v, sem_r)
    g1 = pltpu.async_copy(ffn_hbm.at[idx1_v], rows1_v, sem_r)
    g0.wait()
    w0 = pltpu.async_copy(rows0_v, out_hbm.at[pl.ds(base, HALF)], sem_w)
    g1.wait()
    w1 = pltpu.async_copy(rows1_v, out_hbm.at[pl.ds(base + HALF, HALF)], sem_w)
    w0.wait()
    w1.wait()


def _combine_call(ffn, slot):
    mesh = plsc.VectorSubcoreMesh(core_axis_name="c", subcore_axis_name="s",
                                  num_cores=2, num_subcores=16)
    return pl.kernel(
        _combine_body,
        out_type=jax.ShapeDtypeStruct((N, D), jnp.float32),
        mesh=mesh,
        scratch_types=[
            pltpu.VMEM((HALF,), jnp.int32),
            pltpu.VMEM((HALF,), jnp.int32),
            pltpu.VMEM((HALF, D), jnp.float32),
            pltpu.VMEM((HALF, D), jnp.float32),
            pltpu.SemaphoreType.DMA,
            pltpu.SemaphoreType.DMA,
        ],
    )(ffn, slot)


def kernel(x, Wr, br, W1, b1, W2, b2, Wo1, bo1, Wo2, bo2):
    xf = x.reshape(N, D)
    slot2, ovcnt = _route_call(xf, Wr.T, br.reshape(1, E))
    slot = slot2.reshape(N)
    disp = _dispatch_call(xf, slot)
    ffn = _ffn_call(disp, W1, b1, W2, b2)
    ffn = _ovf_call(ovcnt.reshape(1), disp, ffn, Wo1, bo1, Wo2, bo2)
    out = _combine_call(ffn, slot)
    return out.reshape(B, S, D)


# speedup vs baseline: 1.2098x; 1.0186x over previous
import functools

import jax
import jax.numpy as jnp
from jax import lax
from jax.experimental import pallas as pl
from jax.experimental.pallas import tpu as pltpu
from jax.experimental.pallas import tpu_sc as plsc

B, S, D, F, E = 2, 2048, 768, 3072, 8
CAP = 400
N = B * S
EROWS = E * CAP
RBLK = 400
RB = -(-N // RBLK)
TOT = EROWS + RB * RBLK
BLK = 512
NB = N // BLK
FBLK = 1536
FB = F // FBLK
NW = 32
CHUNK = N // NW


def _route_body(x_ref, wrt_ref, br_ref, slot_ref, ovcnt_ref, base_ref, ovbase_ref):
    i = pl.program_id(0)

    @pl.when(i == 0)
    def _init():
        base_ref[...] = jnp.zeros_like(base_ref)
        ovbase_ref[...] = jnp.zeros_like(ovbase_ref)

    xb = x_ref[...]
    logits = jnp.dot(xb, wrt_ref[...], preferred_element_type=jnp.float32)
    logits = logits + br_ref[...]

    maxv = jnp.max(logits, axis=1, keepdims=True)
    cols = lax.broadcasted_iota(jnp.int32, (BLK, E), 1).astype(jnp.float32)
    eidf = jnp.min(jnp.where(logits >= maxv, cols, float(E)), axis=1,
                   keepdims=True)
    onehot = (cols == eidf).astype(jnp.float32)

    def csum0(a):
        s = 1
        while s < BLK:
            z = jnp.zeros((s, a.shape[1]), a.dtype)
            a = a + jnp.concatenate([z, a[:BLK - s]], axis=0)
            s *= 2
        return a

    csum = csum0(onehot)

    rank = jnp.sum(onehot * (base_ref[...] + csum), axis=1,
                   keepdims=True) - 1.0
    keep = rank < float(CAP)
    ov = jnp.where(keep, 0.0, 1.0)
    ovcsum = csum0(ov)
    ovrank = ovbase_ref[...] + ovcsum - 1.0

    slotf = jnp.where(keep, eidf * float(CAP) + rank,
                      float(EROWS) + ovrank)
    slot_ref[...] = slotf.astype(jnp.int32)

    base_ref[...] = base_ref[...] + jnp.sum(onehot, axis=0, keepdims=True)
    ovbase_ref[...] = ovbase_ref[...] + jnp.sum(ov, axis=0, keepdims=True)

    @pl.when(i == NB - 1)
    def _fin():
        ovcnt_ref[...] = ovbase_ref[...].astype(jnp.int32)


def _route_call(xf, wrt, br2, interpret=False):
    return pl.pallas_call(
        _route_body,
        grid=(NB,),
        in_specs=[
            pl.BlockSpec((BLK, D), lambda i: (i, 0)),
            pl.BlockSpec((D, E), lambda i: (0, 0)),
            pl.BlockSpec((1, E), lambda i: (0, 0)),
        ],
        out_specs=[
            pl.BlockSpec((BLK, 1), lambda i: (i, 0)),
            pl.BlockSpec((1, 1), lambda i: (0, 0)),
        ],
        out_shape=[
            jax.ShapeDtypeStruct((N, 1), jnp.int32),
            jax.ShapeDtypeStruct((1, 1), jnp.int32),
        ],
        scratch_shapes=[
            pltpu.VMEM((1, E), jnp.float32),
            pltpu.VMEM((1, 1), jnp.float32),
        ],
        interpret=interpret,
    )(xf, wrt, br2)


HALF = CHUNK // 2


def _dispatch_body(x_hbm, slot_hbm, out_hbm, idx0_v, idx1_v, rows0_v, rows1_v,
                   sem_r, sem_w):
    wid = lax.axis_index("s") * 2 + lax.axis_index("c")
    base = wid * CHUNK
    pltpu.sync_copy(slot_hbm.at[pl.ds(base, HALF)], idx0_v)
    pltpu.sync_copy(slot_hbm.at[pl.ds(base + HALF, HALF)], idx1_v)
    r0 = pltpu.async_copy(x_hbm.at[pl.ds(base, HALF)], rows0_v, sem_r)
    r1 = pltpu.async_copy(x_hbm.at[pl.ds(base + HALF, HALF)], rows1_v, sem_r)
    r0.wait()
    w0 = pltpu.async_copy(rows0_v, out_hbm.at[idx0_v], sem_w)
    r1.wait()
    w1 = pltpu.async_copy(rows1_v, out_hbm.at[idx1_v], sem_w)
    w0.wait()
    w1.wait()


def _dispatch_call(xf, slot):
    mesh = plsc.VectorSubcoreMesh(core_axis_name="c", subcore_axis_name="s",
                                  num_cores=2, num_subcores=16)
    return pl.kernel(
        _dispatch_body,
        out_type=jax.ShapeDtypeStruct((TOT, D), jnp.float32),
        mesh=mesh,
        scratch_types=[
            pltpu.VMEM((HALF,), jnp.int32),
            pltpu.VMEM((HALF,), jnp.int32),
            pltpu.VMEM((HALF, D), jnp.float32),
            pltpu.VMEM((HALF, D), jnp.float32),
            pltpu.SemaphoreType.DMA,
            pltpu.SemaphoreType.DMA,
        ],
    )(xf, slot)


def _ffn_body(x_ref, w1_ref, b1_ref, w2_ref, b2_ref, out_ref):
    h = jnp.dot(x_ref[...], w1_ref[0], preferred_element_type=jnp.float32)
    h = jnp.maximum(h + b1_ref[0], 0.0)
    out_ref[...] = jnp.dot(h, w2_ref[0],
                           preferred_element_type=jnp.float32) + b2_ref[0]


def _ffn_call(disp, W1, b1, W2, b2, interpret=False):
    return pl.pallas_call(
        _ffn_body,
        grid=(E,),
        in_specs=[
            pl.BlockSpec((CAP, D), lambda e: (e, 0)),
            pl.BlockSpec((1, D, F), lambda e: (e, 0, 0)),
            pl.BlockSpec((1, 1, F), lambda e: (e, 0, 0)),
            pl.BlockSpec((1, F, D), lambda e: (e, 0, 0)),
            pl.BlockSpec((1, 1, D), lambda e: (e, 0, 0)),
        ],
        out_specs=pl.BlockSpec((CAP, D), lambda e: (e, 0)),
        out_shape=jax.ShapeDtypeStruct((TOT, D), jnp.float32),
        interpret=interpret,
    )(disp, W1, b1.reshape(E, 1, F), W2, b2.reshape(E, 1, D))


def _ovf_row_idx(rb, cnt):
    na = (cnt[0] + RBLK - 1) // RBLK
    return (E + jnp.minimum(rb, jnp.maximum(na, 1) - 1), 0)


def _ovf_body(cnt_ref, x_ref, wo1_ref, bo1_ref, wo2_ref, bo2_ref, _, out_ref):
    rb = pl.program_id(0)

    @pl.when(rb * RBLK < cnt_ref[0])
    def _go():
        h = jnp.dot(x_ref[...], wo1_ref[...], preferred_element_type=jnp.float32)
        h = jnp.maximum(h + bo1_ref[...], 0.0)
        out_ref[...] = jnp.dot(h, wo2_ref[...],
                               preferred_element_type=jnp.float32) + bo2_ref[...]


def _ovf_call(ovcnt1, disp, ffn, Wo1, bo1, Wo2, bo2, interpret=False):
    grid_spec = pltpu.PrefetchScalarGridSpec(
        num_scalar_prefetch=1,
        grid=(RB,),
        in_specs=[
            pl.BlockSpec((RBLK, D), _ovf_row_idx),
            pl.BlockSpec((D, F), lambda rb, c: (0, 0)),
            pl.BlockSpec((1, F), lambda rb, c: (0, 0)),
            pl.BlockSpec((F, D), lambda rb, c: (0, 0)),
            pl.BlockSpec((1, D), lambda rb, c: (0, 0)),
            pl.BlockSpec(memory_space=pl.ANY),
        ],
        out_specs=pl.BlockSpec((RBLK, D), _ovf_row_idx),
    )
    return pl.pallas_call(
        _ovf_body,
        grid_spec=grid_spec,
        out_shape=jax.ShapeDtypeStruct((TOT, D), jnp.float32),
        input_output_aliases={6: 0},
        interpret=interpret,
    )(ovcnt1, disp, Wo1, bo1.reshape(1, F), Wo2, bo2.reshape(1, D), ffn)


def _combine_body(ffn_hbm, slot_hbm, out_hbm, idx0_v, idx1_v, rows0_v, rows1_v,
                  sem_r, sem_w):
    wid = lax.axis_index("s") * 2 + lax.axis_index("c")
    base = wid * CHUNK
    pltpu.sync_copy(slot_hbm.at[pl.ds(base, HALF)], idx0_v)
    pltpu.sync_copy(slot_hbm.at[pl.ds(base + HALF, HALF)], idx1_v)
    g0 = pltpu.async_copy(ffn_hbm.at[idx0_v], rows0_v, sem_r)
    g1 = pltpu.async_copy(ffn_hbm.at[idx1_v], rows1_v, sem_r)
    g0.wait()
    w0 = pltpu.async_copy(rows0_v, out_hbm.at[pl.ds(base, HALF)], sem_w)
    g1.wait()
    w1 = pltpu.async_copy(rows1_v, out_hbm.at[pl.ds(base + HALF, HALF)], sem_w)
    w0.wait()
    w1.wait()


def _combine_call(ffn, slot):
    mesh = plsc.VectorSubcoreMesh(core_axis_name="c", subcore_axis_name="s",
                                  num_cores=2, num_subcores=16)
    return pl.kernel(
        _combine_body,
        out_type=jax.ShapeDtypeStruct((N, D), jnp.float32),
        mesh=mesh,
        scratch_types=[
            pltpu.VMEM((HALF,), jnp.int32),
            pltpu.VMEM((HALF,), jnp.int32),
            pltpu.VMEM((HALF, D), jnp.float32),
            pltpu.VMEM((HALF, D), jnp.float32),
            pltpu.SemaphoreType.DMA,
            pltpu.SemaphoreType.DMA,
        ],
    )(ffn, slot)


def kernel(x, Wr, br, W1, b1, W2, b2, Wo1, bo1, Wo2, bo2):
    xf = x.reshape(N, D)
    slot2, ovcnt = _route_call(xf, Wr.T, br.reshape(1, E))
    slot = slot2.reshape(N)
    disp = _dispatch_call(xf, slot)
    ffn = _ffn_call(disp, W1, b1, W2, b2)
    ffn = _ovf_call(ovcnt.reshape(1), disp, ffn, Wo1, bo1, Wo2, bo2)
    out = _combine_call(ffn, slot)
    return out.reshape(B, S, D)


# speedup vs baseline: 1.2121x; 1.0019x over previous
import functools

import jax
import jax.numpy as jnp
from jax import lax
from jax.experimental import pallas as pl
from jax.experimental.pallas import tpu as pltpu
from jax.experimental.pallas import tpu_sc as plsc

B, S, D, F, E = 2, 2048, 768, 3072, 8
CAP = 400
N = B * S
EROWS = E * CAP
RBLK = 128
RB = -(-N // RBLK)
TOT = EROWS + RB * RBLK
OVB0 = EROWS // RBLK
BLK = 512
NB = N // BLK
FBLK = 1536
FB = F // FBLK
NW = 32
CHUNK = N // NW


def _route_body(x_ref, wrt_ref, br_ref, slot_ref, ovcnt_ref, base_ref):
    i = pl.program_id(0)

    @pl.when(i == 0)
    def _init():
        base_ref[...] = jnp.zeros_like(base_ref)

    xb = x_ref[...]
    logits = jnp.dot(xb, wrt_ref[...], preferred_element_type=jnp.float32)
    logits = logits + br_ref[...]

    maxv = jnp.max(logits, axis=1, keepdims=True)
    cols = lax.broadcasted_iota(jnp.int32, (BLK, E), 1).astype(jnp.float32)
    eidf = jnp.min(jnp.where(logits >= maxv, cols, float(E)), axis=1,
                   keepdims=True)
    onehot = (cols == eidf).astype(jnp.float32)

    def csum0(a):
        s = 1
        while s < BLK:
            z = jnp.zeros((s, a.shape[1]), a.dtype)
            a = a + jnp.concatenate([z, a[:BLK - s]], axis=0)
            s *= 2
        return a

    csum = csum0(onehot)

    counts = base_ref[...] + csum
    rank = jnp.sum(onehot * counts, axis=1, keepdims=True) - 1.0
    keep = rank < float(CAP)
    kept_incl = jnp.sum(jnp.minimum(counts, float(CAP)), axis=1,
                        keepdims=True)
    gidx = (jnp.float32(i * BLK)
            + lax.broadcasted_iota(jnp.int32, (BLK, 1), 0).astype(jnp.float32))
    ovrank = gidx - kept_incl

    slotf = jnp.where(keep, eidf * float(CAP) + rank,
                      float(EROWS) + ovrank)
    slot_ref[...] = slotf.astype(jnp.int32)

    base_ref[...] = base_ref[...] + jnp.sum(onehot, axis=0, keepdims=True)

    @pl.when(i == NB - 1)
    def _fin():
        kept_tot = jnp.sum(jnp.minimum(base_ref[...], float(CAP)),
                           axis=1, keepdims=True)
        ovcnt_ref[...] = (float(N) - kept_tot).astype(jnp.int32)


def _route_call(xf, wrt, br2, interpret=False):
    return pl.pallas_call(
        _route_body,
        grid=(NB,),
        in_specs=[
            pl.BlockSpec((BLK, D), lambda i: (i, 0)),
            pl.BlockSpec((D, E), lambda i: (0, 0)),
            pl.BlockSpec((1, E), lambda i: (0, 0)),
        ],
        out_specs=[
            pl.BlockSpec((BLK, 1), lambda i: (i, 0)),
            pl.BlockSpec((1, 1), lambda i: (0, 0)),
        ],
        out_shape=[
            jax.ShapeDtypeStruct((N, 1), jnp.int32),
            jax.ShapeDtypeStruct((1, 1), jnp.int32),
        ],
        scratch_shapes=[
            pltpu.VMEM((1, E), jnp.float32),
        ],
        interpret=interpret,
    )(xf, wrt, br2)


HALF = CHUNK // 2


def _dispatch_body(x_hbm, slot_hbm, out_hbm, idx0_v, idx1_v, rows0_v, rows1_v,
                   sem_r, sem_w):
    wid = lax.axis_index("s") * 2 + lax.axis_index("c")
    base = wid * CHUNK
    pltpu.sync_copy(slot_hbm.at[pl.ds(base, HALF)], idx0_v)
    pltpu.sync_copy(slot_hbm.at[pl.ds(base + HALF, HALF)], idx1_v)
    r0 = pltpu.async_copy(x_hbm.at[pl.ds(base, HALF)], rows0_v, sem_r)
    r1 = pltpu.async_copy(x_hbm.at[pl.ds(base + HALF, HALF)], rows1_v, sem_r)
    r0.wait()
    w0 = pltpu.async_copy(rows0_v, out_hbm.at[idx0_v], sem_w)
    r1.wait()
    w1 = pltpu.async_copy(rows1_v, out_hbm.at[idx1_v], sem_w)
    w0.wait()
    w1.wait()


def _dispatch_call(xf, slot):
    mesh = plsc.VectorSubcoreMesh(core_axis_name="c", subcore_axis_name="s",
                                  num_cores=2, num_subcores=16)
    return pl.kernel(
        _dispatch_body,
        out_type=jax.ShapeDtypeStruct((TOT, D), jnp.float32),
        mesh=mesh,
        scratch_types=[
            pltpu.VMEM((HALF,), jnp.int32),
            pltpu.VMEM((HALF,), jnp.int32),
            pltpu.VMEM((HALF, D), jnp.float32),
            pltpu.VMEM((HALF, D), jnp.float32),
            pltpu.SemaphoreType.DMA,
            pltpu.SemaphoreType.DMA,
        ],
    )(xf, slot)


def _ffn_body(x_ref, w1_ref, b1_ref, w2_ref, b2_ref, out_ref):
    h = jnp.dot(x_ref[...], w1_ref[0], preferred_element_type=jnp.float32)
    h = jnp.maximum(h + b1_ref[0], 0.0)
    out_ref[...] = jnp.dot(h, w2_ref[0],
                           preferred_element_type=jnp.float32) + b2_ref[0]


def _ffn_call(disp, W1, b1, W2, b2, interpret=False):
    return pl.pallas_call(
        _ffn_body,
        grid=(E,),
        in_specs=[
            pl.BlockSpec((CAP, D), lambda e: (e, 0)),
            pl.BlockSpec((1, D, F), lambda e: (e, 0, 0)),
            pl.BlockSpec((1, 1, F), lambda e: (e, 0, 0)),
            pl.BlockSpec((1, F, D), lambda e: (e, 0, 0)),
            pl.BlockSpec((1, 1, D), lambda e: (e, 0, 0)),
        ],
        out_specs=pl.BlockSpec((CAP, D), lambda e: (e, 0)),
        out_shape=jax.ShapeDtypeStruct((TOT, D), jnp.float32),
        interpret=interpret,
    )(disp, W1, b1.reshape(E, 1, F), W2, b2.reshape(E, 1, D))


def _ovf_row_idx(rb, cnt):
    na = (cnt[0] + RBLK - 1) // RBLK
    return (OVB0 + jnp.minimum(rb, jnp.maximum(na, 1) - 1), 0)


def _ovf_body(cnt_ref, x_ref, wo1_ref, bo1_ref, wo2_ref, bo2_ref, _, out_ref):
    rb = pl.program_id(0)

    @pl.when(rb * RBLK < cnt_ref[0])
    def _go():
        h = jnp.dot(x_ref[...], wo1_ref[...], preferred_element_type=jnp.float32)
        h = jnp.maximum(h + bo1_ref[...], 0.0)
        out_ref[...] = jnp.dot(h, wo2_ref[...],
                               preferred_element_type=jnp.float32) + bo2_ref[...]


def _ovf_call(ovcnt1, disp, ffn, Wo1, bo1, Wo2, bo2, interpret=False):
    grid_spec = pltpu.PrefetchScalarGridSpec(
        num_scalar_prefetch=1,
        grid=(RB,),
        in_specs=[
            pl.BlockSpec((RBLK, D), _ovf_row_idx),
            pl.BlockSpec((D, F), lambda rb, c: (0, 0)),
            pl.BlockSpec((1, F), lambda rb, c: (0, 0)),
            pl.BlockSpec((F, D), lambda rb, c: (0, 0)),
            pl.BlockSpec((1, D), lambda rb, c: (0, 0)),
            pl.BlockSpec(memory_space=pl.ANY),
        ],
        out_specs=pl.BlockSpec((RBLK, D), _ovf_row_idx),
    )
    return pl.pallas_call(
        _ovf_body,
        grid_spec=grid_spec,
        out_shape=jax.ShapeDtypeStruct((TOT, D), jnp.float32),
        input_output_aliases={6: 0},
        interpret=interpret,
    )(ovcnt1, disp, Wo1, bo1.reshape(1, F), Wo2, bo2.reshape(1, D), ffn)


def _combine_body(ffn_hbm, slot_hbm, out_hbm, idx0_v, idx1_v, rows0_v, rows1_v,
                  sem_r, sem_w):
    wid = lax.axis_index("s") * 2 + lax.axis_index("c")
    base = wid * CHUNK
    pltpu.sync_copy(slot_hbm.at[pl.ds(base, HALF)], idx0_v)
    pltpu.sync_copy(slot_hbm.at[pl.ds(base + HALF, HALF)], idx1_v)
    g0 = pltpu.async_copy(ffn_hbm.at[idx0_v], rows0_v, sem_r)
    g1 = pltpu.async_copy(ffn_hbm.at[idx1_v], rows1_v, sem_r)
    g0.wait()
    w0 = pltpu.async_copy(rows0_v, out_hbm.at[pl.ds(base, HALF)], sem_w)
    g1.wait()
    w1 = pltpu.async_copy(rows1_v, out_hbm.at[pl.ds(base + HALF, HALF)], sem_w)
    w0.wait()
    w1.wait()


def _combine_call(ffn, slot):
    mesh = plsc.VectorSubcoreMesh(core_axis_name="c", subcore_axis_name="s",
                                  num_cores=2, num_subcores=16)
    return pl.kernel(
        _combine_body,
        out_type=jax.ShapeDtypeStruct((N, D), jnp.float32),
        mesh=mesh,
        scratch_types=[
            pltpu.VMEM((HALF,), jnp.int32),
            pltpu.VMEM((HALF,), jnp.int32),
            pltpu.VMEM((HALF, D), jnp.float32),
            pltpu.VMEM((HALF, D), jnp.float32),
            pltpu.SemaphoreType.DMA,
            pltpu.SemaphoreType.DMA,
        ],
    )(ffn, slot)


def kernel(x, Wr, br, W1, b1, W2, b2, Wo1, bo1, Wo2, bo2):
    xf = x.reshape(N, D)
    slot2, ovcnt = _route_call(xf, Wr.T, br.reshape(1, E))
    slot = slot2.reshape(N)
    disp = _dispatch_call(xf, slot)
    ffn = _ffn_call(disp, W1, b1, W2, b2)
    ffn = _ovf_call(ovcnt.reshape(1), disp, ffn, Wo1, bo1, Wo2, bo2)
    out = _combine_call(ffn, slot)
    return out.reshape(B, S, D)
